# baseline (speedup 1.0000x reference)
import functools

import jax
import jax.numpy as jnp
from jax.experimental import pallas as pl
from jax.experimental.pallas import tpu as pltpu

EMBD = 64
_ARB = pltpu.CompilerParams(dimension_semantics=("arbitrary",))


def _lnk(x, g, b):
    m = jnp.mean(x, axis=-1, keepdims=True)
    v = jnp.mean((x - m) ** 2, axis=-1, keepdims=True)
    return (x - m) * jax.lax.rsqrt(v + 1e-5) * g + b


def _relu(x):
    return jnp.maximum(x, 0.0)


def _dot(a, b):
    return jnp.dot(a, b, preferred_element_type=jnp.float32)


def _row2(v):
    return v.reshape(1, -1)


def _embed_fused(x, emb_p, extras, emit_emb, blk):
    n, din = x.shape
    n_extra = len(extras)

    def body(*refs):
        x_ref, lng, lnb, W1, b1, W2, b2 = refs[:7]
        pos = 7
        ew = refs[pos:pos + 2 * n_extra]
        outs = refs[pos + 2 * n_extra:]
        h = _lnk(x_ref[...], lng[...], lnb[...])
        h = _relu(_dot(h, W1[...]) + b1[...])
        h = _relu(_dot(h, W2[...]) + b2[...])
        oi = 0
        if emit_emb:
            outs[oi][...] = h
            oi += 1
        for k in range(n_extra):
            outs[oi][...] = _dot(h, ew[2 * k][...]) + ew[2 * k + 1][...]
            oi += 1

    full = lambda s: pl.BlockSpec(s, lambda i: (0, 0))
    in_specs = [pl.BlockSpec((blk, din), lambda i: (i, 0)),
                full((1, din)), full((1, din)),
                full((din, EMBD)), full((1, EMBD)),
                full((EMBD, EMBD)), full((1, EMBD))]
    args = [x, _row2(emb_p['lng']), _row2(emb_p['lnb']),
            emb_p['W1'], _row2(emb_p['b1']), emb_p['W2'], _row2(emb_p['b2'])]
    for (W, b) in extras:
        in_specs += [full((EMBD, EMBD)), full((1, EMBD))]
        args += [W, _row2(b)]
    n_out = (1 if emit_emb else 0) + n_extra
    out_specs = [pl.BlockSpec((blk, EMBD), lambda i: (i, 0))] * n_out
    out_shape = [jax.ShapeDtypeStruct((n, EMBD), jnp.float32)] * n_out
    res = pl.pallas_call(
        body, grid=(n // blk,), in_specs=in_specs,
        out_specs=out_specs, out_shape=out_shape, compiler_params=_ARB,
    )(*args)
    return res


def _msg_stage(pre_ab, E, cp, blk):
    n = pre_ab.shape[0]

    def body(p_ref, e_ref, flng, flnb, fW, fb, o_ref):
        x = p_ref[...] + e_ref[...]
        h = _relu(_lnk(x, flng[...], flnb[...]))
        o_ref[...] = _dot(h, fW[...]) + fb[...]

    full = lambda s: pl.BlockSpec(s, lambda i: (0, 0))
    return pl.pallas_call(
        body, grid=(n // blk,),
        in_specs=[pl.BlockSpec((blk, EMBD), lambda i: (i, 0)),
                  pl.BlockSpec((blk, EMBD), lambda i: (i, 0)),
                  full((1, EMBD)), full((1, EMBD)),
                  full((EMBD, EMBD)), full((1, EMBD))],
        out_specs=pl.BlockSpec((blk, EMBD), lambda i: (i, 0)),
        out_shape=jax.ShapeDtypeStruct((n, EMBD), jnp.float32),
        compiler_params=_ARB,
    )(pre_ab, E, _row2(cp['flng']), _row2(cp['flnb']), cp['fW'], _row2(cp['fb']))


def _node_stage(agg, emb, cp, tail, blk):
    n = agg.shape[0]
    oW1a = cp['oW1'][:EMBD]
    oW1b = cp['oW1'][EMBD:]
    mode, tail_args = tail

    def body(*refs):
        (agg_ref, emb_ref, plng, plnb, w1a, w1b, ob1, oW2, ob2) = refs[:9]
        rest = refs[9:]
        a = _lnk(agg_ref[...], plng[...], plnb[...])
        h = _relu(_dot(a, w1a[...]) + _dot(emb_ref[...], w1b[...]) + ob1[...])
        nn = _dot(h, oW2[...]) + ob2[...]
        if mode == 'proj':
            (rW, o_ref) = rest
            o_ref[...] = _dot(nn, rW[...])
        else:
            (W1, b1, w2, o_ref) = rest
            r = _relu(_dot(nn, W1[...]) + b1[...])
            o_ref[...] = jnp.sum(r * w2[...], axis=-1, keepdims=True)

    full = lambda s: pl.BlockSpec(s, lambda i: (0, 0))
    in_specs = [pl.BlockSpec((blk, EMBD), lambda i: (i, 0)),
                pl.BlockSpec((blk, EMBD), lambda i: (i, 0)),
                full((1, EMBD)), full((1, EMBD)),
                full((EMBD, EMBD)), full((EMBD, EMBD)), full((1, EMBD)),
                full((EMBD, EMBD)), full((1, EMBD))]
    args = [agg, emb, _row2(cp['plng']), _row2(cp['plnb']),
            oW1a, oW1b, _row2(cp['ob1']), cp['oW2'], _row2(cp['ob2'])]
    if mode == 'proj':
        (rW,) = tail_args
        in_specs += [full((EMBD, EMBD))]
        args += [rW]
        out_cols = EMBD
    else:
        (W1, b1, W2) = tail_args
        in_specs += [full((EMBD, EMBD)), full((1, EMBD)), full((1, EMBD))]
        args += [W1, _row2(b1), W2.reshape(1, EMBD)]
        out_cols = 1
    return pl.pallas_call(
        body, grid=(n // blk,), in_specs=in_specs,
        out_specs=pl.BlockSpec((blk, out_cols), lambda i: (i, 0)),
        out_shape=jax.ShapeDtypeStruct((n, out_cols), jnp.float32),
        compiler_params=_ARB,
    )(*args)


def _gather_add(A, B, tgt, src):
    return A[tgt] + B[src]


def _scatter_add(msg, tgt, n_nodes):
    return jnp.zeros((n_nodes, EMBD), jnp.float32).at[tgt].add(msg)


def kernel(constraint_features, edge_indices, edge_features, variable_features, params):
    NC = constraint_features.shape[0]
    NV = variable_features.shape[0]
    NE = edge_features.shape[0]
    p1 = params['conv1']
    p2 = params['conv2']

    ei0 = edge_indices[0]
    ei1 = edge_indices[1]

    cemb, A1 = _embed_fused(constraint_features, params['ant'],
                            [(p1['lW'], p1['lb'])], True, blk=1000)
    vemb, B1, A2 = _embed_fused(variable_features, params['var'],
                                [(p1['rW'], jnp.zeros((EMBD,), jnp.float32)),
                                 (p2['lW'], p2['lb'])], True, blk=1000)
    E1, E2 = _embed_fused(edge_features, params['edge'],
                          [(p1['eW'], jnp.zeros((EMBD,), jnp.float32)),
                           (p2['eW'], jnp.zeros((EMBD,), jnp.float32))],
                          False, blk=8000)

    pre1 = _gather_add(A1, B1, ei0, ei1)
    msg1 = _msg_stage(pre1, E1, p1, blk=8000)
    agg1 = _scatter_add(msg1, ei0, NC)
    B2 = _node_stage(agg1, cemb, p1, ('proj', (p2['rW'],)), blk=1000)

    pre2 = _gather_add(A2, B2, ei1, ei0)
    msg2 = _msg_stage(pre2, E2, p2, blk=8000)
    agg2 = _scatter_add(msg2, ei1, NV)
    out = _node_stage(agg2, vemb, p2,
                      ('head', (params['out']['W1'], params['out']['b1'],
                                params['out']['W2'])), blk=1000)
    return out[:, 0]
